# baseline (speedup 1.0000x reference)
import functools

import jax
import jax.numpy as jnp
from jax import lax
from jax.experimental import pallas as pl
from jax.experimental.pallas import tpu as pltpu
from jax.experimental.pallas import tpu_sc as plsc

N = 10000
F = 128
C = 47
CP = 48
NC, NS = 2, 16
NW = NC * NS
NPAD = 10240
CHUNK = 128
EPT = 10240
NCHUNK = EPT // CHUNK
GCH = 80
NBUF = 4
NGCH = EPT // GCH
K0 = 64
K1 = 2 * NGCH - K0
EPAD = NW * EPT
R16 = NPAD // NS
RBLK = 512
GRID = NPAD // RBLK


def _sc_mesh():
  return plsc.VectorSubcoreMesh(
      core_axis_name="c", subcore_axis_name="s",
      num_cores=NC, num_subcores=NS)


def _make_sc_agg(width):
  out_type = jax.ShapeDtypeStruct((2 * NPAD, width), jnp.float32)
  scratch = (
      pltpu.VMEM_SHARED((NPAD, width), jnp.float32),
  ) + tuple(
      pltpu.VMEM((GCH,), jnp.int32) for _ in range(2 * NBUF)
  ) + tuple(
      pltpu.VMEM((GCH, width), jnp.float32) for _ in range(NBUF)
  ) + tuple(
      pltpu.SemaphoreType.DMA for _ in range(NBUF)
  )

  def body(table, srcp, dstp, zrow, s_out, acc_sh, *bufs):
    srcs = bufs[0:2 * NBUF:2]
    dsts = bufs[1:2 * NBUF:2]
    rows = bufs[2 * NBUF:3 * NBUF]
    sems = bufs[3 * NBUF:4 * NBUF]
    cid = lax.axis_index("c")
    sid = lax.axis_index("s")
    wid = sid * NC + cid

    pltpu.sync_copy(zrow.at[pl.ds(0, GCH)], rows[0])

    def zstep(k, carry):
      pltpu.sync_copy(rows[0], acc_sh.at[pl.ds(sid * R16 + k * GCH, GCH)])
      return carry

    lax.fori_loop(0, R16 // GCH, zstep, 0)
    plsc.subcore_barrier()

    ngch = jnp.where(cid == 0, K0, K1)
    base0 = jnp.where(cid == 0, sid * K0, NS * K0 + sid * K1) * GCH

    def load_idx(i, sv, dv):
      base = base0 + i * GCH
      pltpu.sync_copy(srcp.at[pl.ds(base, GCH)], sv)
      pltpu.sync_copy(dstp.at[pl.ds(base, GCH)], dv)

    for b in range(NBUF):
      load_idx(b, srcs[b], dsts[b])
      pltpu.async_copy(table.at[srcs[b]], rows[b], sems[b])

    def group(g, carry):
      for b in range(NBUF):
        i = NBUF * g + b
        pltpu.make_async_copy(table.at[srcs[b]], rows[b], sems[b]).wait()
        pltpu.sync_copy(rows[b], acc_sh.at[dsts[b]], add=True)
        load_idx(i + NBUF, srcs[b], dsts[b])
        pltpu.async_copy(table.at[srcs[b]], rows[b], sems[b])
      return carry

    lax.fori_loop(0, ngch // NBUF - 1, group, 0)

    for b in range(NBUF):
      pltpu.make_async_copy(table.at[srcs[b]], rows[b], sems[b]).wait()
      pltpu.sync_copy(rows[b], acc_sh.at[dsts[b]], add=True)

    plsc.subcore_barrier()

    def wstep(k, carry):
      off = sid * R16 + k * GCH
      pltpu.sync_copy(acc_sh.at[pl.ds(off, GCH)], rows[0])
      pltpu.sync_copy(rows[0], s_out.at[pl.ds(cid * NPAD + off, GCH)])
      return carry

    lax.fori_loop(0, R16 // GCH, wstep, 0)

  return pl.kernel(body, out_type=out_type, mesh=_sc_mesh(),
                   scratch_types=scratch)


def _make_sc_deg():
  out_type = jax.ShapeDtypeStruct((2 * NPAD, F), jnp.float32)
  scratch = (
      pltpu.VMEM_SHARED((NPAD, F), jnp.float32),
      pltpu.VMEM((CHUNK,), jnp.int32),
      pltpu.VMEM((CHUNK, F), jnp.float32),
  )

  def body(dstp, ones_h, d_out, deg_sh, dst_v, rows_v):
    cid = lax.axis_index("c")
    sid = lax.axis_index("s")
    wid = sid * NC + cid

    pltpu.sync_copy(ones_h.at[1], rows_v)

    def zstep(k, carry):
      pltpu.sync_copy(rows_v, deg_sh.at[pl.ds(sid * R16 + k * CHUNK, CHUNK)])
      return carry

    lax.fori_loop(0, R16 // CHUNK, zstep, 0)
    pltpu.sync_copy(ones_h.at[0], rows_v)
    plsc.subcore_barrier()

    base0 = wid * EPT

    def step(i, carry):
      pltpu.sync_copy(dstp.at[pl.ds(base0 + i * CHUNK, CHUNK)], dst_v)
      pltpu.sync_copy(rows_v, deg_sh.at[dst_v], add=True)
      return carry

    lax.fori_loop(0, NCHUNK, step, 0)
    plsc.subcore_barrier()

    def wstep(k, carry):
      off = sid * R16 + k * CHUNK
      pltpu.sync_copy(deg_sh.at[pl.ds(off, CHUNK)], rows_v)
      pltpu.sync_copy(rows_v, d_out.at[pl.ds(cid * NPAD + off, CHUNK)])
      return carry

    lax.fori_loop(0, R16 // CHUNK, wstep, 0)

  return pl.kernel(body, out_type=out_type, mesh=_sc_mesh(),
                   scratch_types=scratch)


def _row_spec(w):
  return pl.BlockSpec((RBLK, w), lambda i: (i, 0))


def _full_spec(r, w):
  return pl.BlockSpec((r, w), lambda i: (0, 0))


def _inv_deg(d0, d1):
  return 1.0 / jnp.maximum(d0[:, 0:1] + d1[:, 0:1], 1.0)


def _tc_layer1(s0, s1, d0, d1, x, wlT, bl, wrT, out):
  agg = (s0[...] + s1[...]) * _inv_deg(d0, d1)
  y = jnp.dot(agg, wlT[...], preferred_element_type=jnp.float32) + bl[...]
  y = y + jnp.dot(x[...], wrT[...], preferred_element_type=jnp.float32)
  out[...] = jnp.maximum(y, 0.0)


def _tc_final(s0, s1, d0, d1, h2, wl3T, wr3T, bl3, out):
  agg = (s0[...] + s1[...]) * _inv_deg(d0, d1)
  logits = jnp.dot(agg, wl3T[...], preferred_element_type=jnp.float32)
  logits = logits + jnp.dot(h2[...], wr3T[...],
                            preferred_element_type=jnp.float32) + bl3[...]
  mask = lax.broadcasted_iota(jnp.int32, (1, CP), 1) < C
  masked = jnp.where(mask, logits, -jnp.inf)
  m = jnp.max(masked, axis=1, keepdims=True)
  ex = jnp.where(mask, jnp.exp(logits - m), 0.0)
  lse = jnp.log(jnp.sum(ex, axis=1, keepdims=True)) + m
  out[...] = logits - lse


def kernel(x, edge_index, Wl1, bl1, Wr1, Wl2, bl2, Wr2, Wl3, bl3, Wr3):
  e = edge_index.shape[1]
  pad = EPAD - e
  src = jnp.concatenate([edge_index[0], jnp.zeros((pad,), jnp.int32)])
  dst = jnp.concatenate([edge_index[1], jnp.full((pad,), N, jnp.int32)])

  zacc = jnp.zeros((NPAD, F), jnp.float32)
  const_rows = jnp.stack([jnp.ones((CHUNK, F), jnp.float32),
                          jnp.zeros((CHUNK, F), jnp.float32)])

  wl1T, wr1T = Wl1.T, Wr1.T
  wl2T, wr2T = Wl2.T, Wr2.T
  wl3T = jnp.pad(Wl3.T, ((0, 0), (0, CP - C)))
  wr3T = jnp.pad(Wr3.T, ((0, 0), (0, CP - C)))
  bl3p = jnp.pad(bl3[None, :], ((0, 0), (0, CP - C)))
  bl1r = bl1[None, :]
  bl2r = bl2[None, :]

  agg128 = _make_sc_agg(F)

  dd = _make_sc_deg()(dst, const_rows)
  d0, d1 = dd[:NPAD], dd[NPAD:]

  ss = agg128(x, src, dst, zacc)
  s10, s11 = ss[:NPAD], ss[NPAD:]

  h1 = pl.pallas_call(
      _tc_layer1,
      grid=(GRID,),
      in_specs=[_row_spec(F), _row_spec(F), _row_spec(F), _row_spec(F),
                _row_spec(F), _full_spec(F, F), _full_spec(1, F),
                _full_spec(F, F)],
      out_specs=_row_spec(F),
      out_shape=jax.ShapeDtypeStruct((N, F), jnp.float32),
  )(s10, s11, d0, d1, x, wl1T, bl1r, wr1T)

  ss = agg128(h1, src, dst, zacc)
  s20, s21 = ss[:NPAD], ss[NPAD:]

  h2 = pl.pallas_call(
      _tc_layer1,
      grid=(GRID,),
      in_specs=[_row_spec(F), _row_spec(F), _row_spec(F), _row_spec(F),
                _row_spec(F), _full_spec(F, F), _full_spec(1, F),
                _full_spec(F, F)],
      out_specs=_row_spec(F),
      out_shape=jax.ShapeDtypeStruct((N, F), jnp.float32),
  )(s20, s21, d0, d1, h1, wl2T, bl2r, wr2T)

  ss = agg128(h2, src, dst, zacc)
  s30, s31 = ss[:NPAD], ss[NPAD:]

  outp = pl.pallas_call(
      _tc_final,
      grid=(GRID,),
      in_specs=[_row_spec(F), _row_spec(F), _row_spec(F), _row_spec(F),
                _row_spec(F), _full_spec(F, CP), _full_spec(F, CP),
                _full_spec(1, CP)],
      out_specs=_row_spec(CP),
      out_shape=jax.ShapeDtypeStruct((N, CP), jnp.float32),
  )(s30, s31, d0, d1, h2, wl3T, wr3T, bl3p)

  return outp[:, :C]


# speedup vs baseline: 1.2221x; 1.2221x over previous
import functools

import jax
import jax.numpy as jnp
from jax import lax
from jax.experimental import pallas as pl
from jax.experimental.pallas import tpu as pltpu
from jax.experimental.pallas import tpu_sc as plsc

N = 10000
F = 128
C = 47
CP = 48
NC, NS = 2, 16
NW = NC * NS
NPAD = 10240
CHUNK = 128
EPT = 10240
NCHUNK = EPT // CHUNK
GCH = 80
NBUF = 4
NGCH = EPT // GCH
K0 = 192
K1 = 2 * NGCH - K0
EPAD = NW * EPT
R16 = NPAD // NS
RBLK = 512
GRID = NPAD // RBLK


def _sc_mesh():
  return plsc.VectorSubcoreMesh(
      core_axis_name="c", subcore_axis_name="s",
      num_cores=NC, num_subcores=NS)


def _make_sc_agg(width):
  out_type = jax.ShapeDtypeStruct((2 * NPAD, width), jnp.float32)
  scratch = (
      pltpu.VMEM_SHARED((NPAD, width), jnp.float32),
  ) + tuple(
      pltpu.VMEM((GCH,), jnp.int32) for _ in range(2 * NBUF)
  ) + tuple(
      pltpu.VMEM((GCH, width), jnp.float32) for _ in range(NBUF)
  ) + tuple(
      pltpu.SemaphoreType.DMA for _ in range(NBUF)
  )

  def body(table, srcp, dstp, zrow, s_out, acc_sh, *bufs):
    srcs = bufs[0:2 * NBUF:2]
    dsts = bufs[1:2 * NBUF:2]
    rows = bufs[2 * NBUF:3 * NBUF]
    sems = bufs[3 * NBUF:4 * NBUF]
    cid = lax.axis_index("c")
    sid = lax.axis_index("s")
    wid = sid * NC + cid

    pltpu.sync_copy(zrow.at[pl.ds(0, GCH)], rows[0])

    def zstep(k, carry):
      pltpu.sync_copy(rows[0], acc_sh.at[pl.ds(sid * R16 + k * GCH, GCH)])
      return carry

    lax.fori_loop(0, R16 // GCH, zstep, 0)
    plsc.subcore_barrier()

    ngch = jnp.where(cid == 0, K0, K1)
    base0 = jnp.where(cid == 0, sid * K0, NS * K0 + sid * K1) * GCH

    def load_idx(i, sv, dv):
      base = base0 + i * GCH
      pltpu.sync_copy(srcp.at[pl.ds(base, GCH)], sv)
      pltpu.sync_copy(dstp.at[pl.ds(base, GCH)], dv)

    for b in range(NBUF):
      load_idx(b, srcs[b], dsts[b])
      pltpu.async_copy(table.at[srcs[b]], rows[b], sems[b])

    def group(g, carry):
      for b in range(NBUF):
        i = NBUF * g + b
        pltpu.make_async_copy(table.at[srcs[b]], rows[b], sems[b]).wait()
        pltpu.sync_copy(rows[b], acc_sh.at[dsts[b]], add=True)
        load_idx(i + NBUF, srcs[b], dsts[b])
        pltpu.async_copy(table.at[srcs[b]], rows[b], sems[b])
      return carry

    lax.fori_loop(0, ngch // NBUF - 1, group, 0)

    for b in range(NBUF):
      pltpu.make_async_copy(table.at[srcs[b]], rows[b], sems[b]).wait()
      pltpu.sync_copy(rows[b], acc_sh.at[dsts[b]], add=True)

    plsc.subcore_barrier()

    def wstep(k, carry):
      off = sid * R16 + k * GCH
      pltpu.sync_copy(acc_sh.at[pl.ds(off, GCH)], rows[0])
      pltpu.sync_copy(rows[0], s_out.at[pl.ds(cid * NPAD + off, GCH)])
      return carry

    lax.fori_loop(0, R16 // GCH, wstep, 0)

  return pl.kernel(body, out_type=out_type, mesh=_sc_mesh(),
                   scratch_types=scratch)


def _make_sc_deg():
  out_type = jax.ShapeDtypeStruct((2 * NPAD, F), jnp.float32)
  scratch = (
      pltpu.VMEM_SHARED((NPAD, F), jnp.float32),
      pltpu.VMEM((CHUNK,), jnp.int32),
      pltpu.VMEM((CHUNK, F), jnp.float32),
  )

  def body(dstp, ones_h, d_out, deg_sh, dst_v, rows_v):
    cid = lax.axis_index("c")
    sid = lax.axis_index("s")
    wid = sid * NC + cid

    pltpu.sync_copy(ones_h.at[1], rows_v)

    def zstep(k, carry):
      pltpu.sync_copy(rows_v, deg_sh.at[pl.ds(sid * R16 + k * CHUNK, CHUNK)])
      return carry

    lax.fori_loop(0, R16 // CHUNK, zstep, 0)
    pltpu.sync_copy(ones_h.at[0], rows_v)
    plsc.subcore_barrier()

    base0 = wid * EPT

    def step(i, carry):
      pltpu.sync_copy(dstp.at[pl.ds(base0 + i * CHUNK, CHUNK)], dst_v)
      pltpu.sync_copy(rows_v, deg_sh.at[dst_v], add=True)
      return carry

    lax.fori_loop(0, NCHUNK, step, 0)
    plsc.subcore_barrier()

    def wstep(k, carry):
      off = sid * R16 + k * CHUNK
      pltpu.sync_copy(deg_sh.at[pl.ds(off, CHUNK)], rows_v)
      pltpu.sync_copy(rows_v, d_out.at[pl.ds(cid * NPAD + off, CHUNK)])
      return carry

    lax.fori_loop(0, R16 // CHUNK, wstep, 0)

  return pl.kernel(body, out_type=out_type, mesh=_sc_mesh(),
                   scratch_types=scratch)


def _row_spec(w):
  return pl.BlockSpec((RBLK, w), lambda i: (i, 0))


def _full_spec(r, w):
  return pl.BlockSpec((r, w), lambda i: (0, 0))


def _inv_deg(d0, d1):
  return 1.0 / jnp.maximum(d0[:, 0:1] + d1[:, 0:1], 1.0)


def _tc_layer1(s0, s1, d0, d1, x, wlT, bl, wrT, out):
  agg = (s0[...] + s1[...]) * _inv_deg(d0, d1)
  y = jnp.dot(agg, wlT[...], preferred_element_type=jnp.float32) + bl[...]
  y = y + jnp.dot(x[...], wrT[...], preferred_element_type=jnp.float32)
  out[...] = jnp.maximum(y, 0.0)


def _tc_final(s0, s1, d0, d1, h2, wl3T, wr3T, bl3, out):
  agg = (s0[...] + s1[...]) * _inv_deg(d0, d1)
  logits = jnp.dot(agg, wl3T[...], preferred_element_type=jnp.float32)
  logits = logits + jnp.dot(h2[...], wr3T[...],
                            preferred_element_type=jnp.float32) + bl3[...]
  mask = lax.broadcasted_iota(jnp.int32, (1, CP), 1) < C
  masked = jnp.where(mask, logits, -jnp.inf)
  m = jnp.max(masked, axis=1, keepdims=True)
  ex = jnp.where(mask, jnp.exp(logits - m), 0.0)
  lse = jnp.log(jnp.sum(ex, axis=1, keepdims=True)) + m
  out[...] = logits - lse


def kernel(x, edge_index, Wl1, bl1, Wr1, Wl2, bl2, Wr2, Wl3, bl3, Wr3):
  e = edge_index.shape[1]
  pad = EPAD - e
  src = jnp.concatenate([edge_index[0], jnp.zeros((pad,), jnp.int32)])
  dst = jnp.concatenate([edge_index[1], jnp.full((pad,), N, jnp.int32)])

  zacc = jnp.zeros((NPAD, F), jnp.float32)
  const_rows = jnp.stack([jnp.ones((CHUNK, F), jnp.float32),
                          jnp.zeros((CHUNK, F), jnp.float32)])

  wl1T, wr1T = Wl1.T, Wr1.T
  wl2T, wr2T = Wl2.T, Wr2.T
  wl3T = jnp.pad(Wl3.T, ((0, 0), (0, CP - C)))
  wr3T = jnp.pad(Wr3.T, ((0, 0), (0, CP - C)))
  bl3p = jnp.pad(bl3[None, :], ((0, 0), (0, CP - C)))
  bl1r = bl1[None, :]
  bl2r = bl2[None, :]

  agg128 = _make_sc_agg(F)

  dd = _make_sc_deg()(dst, const_rows)
  d0, d1 = dd[:NPAD], dd[NPAD:]

  ss = agg128(x, src, dst, zacc)
  s10, s11 = ss[:NPAD], ss[NPAD:]

  h1 = pl.pallas_call(
      _tc_layer1,
      grid=(GRID,),
      in_specs=[_row_spec(F), _row_spec(F), _row_spec(F), _row_spec(F),
                _row_spec(F), _full_spec(F, F), _full_spec(1, F),
                _full_spec(F, F)],
      out_specs=_row_spec(F),
      out_shape=jax.ShapeDtypeStruct((N, F), jnp.float32),
  )(s10, s11, d0, d1, x, wl1T, bl1r, wr1T)

  ss = agg128(h1, src, dst, zacc)
  s20, s21 = ss[:NPAD], ss[NPAD:]

  h2 = pl.pallas_call(
      _tc_layer1,
      grid=(GRID,),
      in_specs=[_row_spec(F), _row_spec(F), _row_spec(F), _row_spec(F),
                _row_spec(F), _full_spec(F, F), _full_spec(1, F),
                _full_spec(F, F)],
      out_specs=_row_spec(F),
      out_shape=jax.ShapeDtypeStruct((N, F), jnp.float32),
  )(s20, s21, d0, d1, h1, wl2T, bl2r, wr2T)

  ss = agg128(h2, src, dst, zacc)
  s30, s31 = ss[:NPAD], ss[NPAD:]

  outp = pl.pallas_call(
      _tc_final,
      grid=(GRID,),
      in_specs=[_row_spec(F), _row_spec(F), _row_spec(F), _row_spec(F),
                _row_spec(F), _full_spec(F, CP), _full_spec(F, CP),
                _full_spec(1, CP)],
      out_specs=_row_spec(CP),
      out_shape=jax.ShapeDtypeStruct((N, CP), jnp.float32),
  )(s30, s31, d0, d1, h2, wl3T, wr3T, bl3p)

  return outp[:, :C]


# speedup vs baseline: 1.2224x; 1.0002x over previous
import jax
import jax.numpy as jnp
from jax import lax
from jax.experimental import pallas as pl
from jax.experimental.pallas import tpu as pltpu
from jax.experimental.pallas import tpu_sc as plsc

N = 10000
F = 128
C = 47
CP = 48
NC, NS = 2, 16
NW = NC * NS
NPAD = 10240
CHUNK = 128
EPT = 10240
NCHUNK = EPT // CHUNK
GCH = 80
NBUF = 4
NGCH = EPT // GCH
K0 = 192
K1 = 2 * NGCH - K0
EPAD = NW * EPT
R16 = NPAD // NS
RBLK = 512
GRID = NPAD // RBLK


def _sc_mesh():
  return plsc.VectorSubcoreMesh(
      core_axis_name="c", subcore_axis_name="s",
      num_cores=NC, num_subcores=NS)


def _make_sc_agg(width):
  out_type = jax.ShapeDtypeStruct((2 * NPAD, width), jnp.float32)
  scratch = (
      pltpu.VMEM_SHARED((NPAD, width), jnp.float32),
  ) + tuple(
      pltpu.VMEM((GCH,), jnp.int32) for _ in range(2 * NBUF)
  ) + tuple(
      pltpu.VMEM((GCH, width), jnp.float32) for _ in range(NBUF)
  ) + tuple(
      pltpu.SemaphoreType.DMA for _ in range(NBUF)
  )

  def body(table, srcp, dstp, zrow, s_out, acc_sh, *bufs):
    srcs = bufs[0:2 * NBUF:2]
    dsts = bufs[1:2 * NBUF:2]
    rows = bufs[2 * NBUF:3 * NBUF]
    sems = bufs[3 * NBUF:4 * NBUF]
    cid = lax.axis_index("c")
    sid = lax.axis_index("s")
    wid = sid * NC + cid

    pltpu.sync_copy(zrow.at[pl.ds(0, GCH)], rows[0])

    def zstep(k, carry):
      pltpu.sync_copy(rows[0], acc_sh.at[pl.ds(sid * R16 + k * GCH, GCH)])
      return carry

    lax.fori_loop(0, R16 // GCH, zstep, 0)
    plsc.subcore_barrier()

    ngch = jnp.where(cid == 0, K0, K1)
    base0 = jnp.where(cid == 0, sid * K0, NS * K0 + sid * K1) * GCH

    def load_idx(i, sv, dv):
      base = base0 + i * GCH
      pltpu.sync_copy(srcp.at[pl.ds(base, GCH)], sv)
      pltpu.sync_copy(dstp.at[pl.ds(base, GCH)], dv)

    for b in range(NBUF):
      load_idx(b, srcs[b], dsts[b])
      pltpu.async_copy(table.at[srcs[b]], rows[b], sems[b])

    def group(g, carry):
      for b in range(NBUF):
        i = NBUF * g + b
        pltpu.make_async_copy(table.at[srcs[b]], rows[b], sems[b]).wait()
        pltpu.sync_copy(rows[b], acc_sh.at[dsts[b]], add=True)
        load_idx(i + NBUF, srcs[b], dsts[b])
        pltpu.async_copy(table.at[srcs[b]], rows[b], sems[b])
      return carry

    lax.fori_loop(0, ngch // NBUF - 1, group, 0)

    for b in range(NBUF):
      pltpu.make_async_copy(table.at[srcs[b]], rows[b], sems[b]).wait()
      pltpu.sync_copy(rows[b], acc_sh.at[dsts[b]], add=True)

    plsc.subcore_barrier()

    def wstep(k, carry):
      off = sid * R16 + k * GCH
      pltpu.sync_copy(acc_sh.at[pl.ds(off, GCH)], rows[0])
      pltpu.sync_copy(rows[0], s_out.at[pl.ds(cid * NPAD + off, GCH)])
      return carry

    lax.fori_loop(0, R16 // GCH, wstep, 0)

  return pl.kernel(body, out_type=out_type, mesh=_sc_mesh(),
                   scratch_types=scratch)


def _make_sc_deg():
  out_type = jax.ShapeDtypeStruct((2 * NPAD, F), jnp.float32)
  scratch = (
      pltpu.VMEM_SHARED((NPAD, F), jnp.float32),
      pltpu.VMEM((CHUNK,), jnp.int32),
      pltpu.VMEM((CHUNK, F), jnp.float32),
  )

  def body(dstp, ones_h, d_out, deg_sh, dst_v, rows_v):
    cid = lax.axis_index("c")
    sid = lax.axis_index("s")
    wid = sid * NC + cid

    pltpu.sync_copy(ones_h.at[1], rows_v)

    def zstep(k, carry):
      pltpu.sync_copy(rows_v, deg_sh.at[pl.ds(sid * R16 + k * CHUNK, CHUNK)])
      return carry

    lax.fori_loop(0, R16 // CHUNK, zstep, 0)
    pltpu.sync_copy(ones_h.at[0], rows_v)
    plsc.subcore_barrier()

    base0 = wid * EPT

    def step(i, carry):
      pltpu.sync_copy(dstp.at[pl.ds(base0 + i * CHUNK, CHUNK)], dst_v)
      pltpu.sync_copy(rows_v, deg_sh.at[dst_v], add=True)
      return carry

    lax.fori_loop(0, NCHUNK, step, 0)
    plsc.subcore_barrier()

    def wstep(k, carry):
      off = sid * R16 + k * CHUNK
      pltpu.sync_copy(deg_sh.at[pl.ds(off, CHUNK)], rows_v)
      pltpu.sync_copy(rows_v, d_out.at[pl.ds(cid * NPAD + off, CHUNK)])
      return carry

    lax.fori_loop(0, R16 // CHUNK, wstep, 0)

  return pl.kernel(body, out_type=out_type, mesh=_sc_mesh(),
                   scratch_types=scratch)


def _row_spec(w):
  return pl.BlockSpec((RBLK, w), lambda i: (i, 0))


def _full_spec(r, w):
  return pl.BlockSpec((r, w), lambda i: (0, 0))


def _inv_deg(d0, d1):
  return 1.0 / jnp.maximum(d0[:, 0:1] + d1[:, 0:1], 1.0)


def _tc_layer1(s0, s1, d0, d1, x, wlT, bl, wrT, out):
  agg = (s0[...] + s1[...]) * _inv_deg(d0, d1)
  y = jnp.dot(agg, wlT[...], preferred_element_type=jnp.float32) + bl[...]
  y = y + jnp.dot(x[...], wrT[...], preferred_element_type=jnp.float32)
  out[...] = jnp.maximum(y, 0.0)


def _tc_final(s0, s1, d0, d1, h2, wl3T, wr3T, bl3, out):
  agg = (s0[...] + s1[...]) * _inv_deg(d0, d1)
  logits = jnp.dot(agg, wl3T[...], preferred_element_type=jnp.float32)
  logits = logits + jnp.dot(h2[...], wr3T[...],
                            preferred_element_type=jnp.float32) + bl3[...]
  mask = lax.broadcasted_iota(jnp.int32, (1, CP), 1) < C
  masked = jnp.where(mask, logits, -jnp.inf)
  m = jnp.max(masked, axis=1, keepdims=True)
  ex = jnp.where(mask, jnp.exp(logits - m), 0.0)
  lse = jnp.log(jnp.sum(ex, axis=1, keepdims=True)) + m
  out[...] = logits - lse


def kernel(x, edge_index, Wl1, bl1, Wr1, Wl2, bl2, Wr2, Wl3, bl3, Wr3):
  e = edge_index.shape[1]
  pad = EPAD - e
  src = jnp.concatenate([edge_index[0], jnp.zeros((pad,), jnp.int32)])
  dst = jnp.concatenate([edge_index[1], jnp.full((pad,), N, jnp.int32)])

  zacc = jnp.zeros((NPAD, F), jnp.float32)
  const_rows = jnp.stack([jnp.ones((CHUNK, F), jnp.float32),
                          jnp.zeros((CHUNK, F), jnp.float32)])

  wl1T, wr1T = Wl1.T, Wr1.T
  wl2T, wr2T = Wl2.T, Wr2.T
  wl3T = jnp.pad(Wl3.T, ((0, 0), (0, CP - C)))
  wr3T = jnp.pad(Wr3.T, ((0, 0), (0, CP - C)))
  bl3p = jnp.pad(bl3[None, :], ((0, 0), (0, CP - C)))
  bl1r = bl1[None, :]
  bl2r = bl2[None, :]

  agg128 = _make_sc_agg(F)

  dd = _make_sc_deg()(dst, const_rows)
  d0, d1 = dd[:NPAD], dd[NPAD:]

  ss = agg128(x, src, dst, zacc)
  s10, s11 = ss[:NPAD], ss[NPAD:]

  h1 = pl.pallas_call(
      _tc_layer1,
      grid=(GRID,),
      in_specs=[_row_spec(F), _row_spec(F), _row_spec(F), _row_spec(F),
                _row_spec(F), _full_spec(F, F), _full_spec(1, F),
                _full_spec(F, F)],
      out_specs=_row_spec(F),
      out_shape=jax.ShapeDtypeStruct((N, F), jnp.float32),
  )(s10, s11, d0, d1, x, wl1T, bl1r, wr1T)

  ss = agg128(h1, src, dst, zacc)
  s20, s21 = ss[:NPAD], ss[NPAD:]

  h2 = pl.pallas_call(
      _tc_layer1,
      grid=(GRID,),
      in_specs=[_row_spec(F), _row_spec(F), _row_spec(F), _row_spec(F),
                _row_spec(F), _full_spec(F, F), _full_spec(1, F),
                _full_spec(F, F)],
      out_specs=_row_spec(F),
      out_shape=jax.ShapeDtypeStruct((N, F), jnp.float32),
  )(s20, s21, d0, d1, h1, wl2T, bl2r, wr2T)

  ss = agg128(h2, src, dst, zacc)
  s30, s31 = ss[:NPAD], ss[NPAD:]

  outp = pl.pallas_call(
      _tc_final,
      grid=(GRID,),
      in_specs=[_row_spec(F), _row_spec(F), _row_spec(F), _row_spec(F),
                _row_spec(F), _full_spec(F, CP), _full_spec(F, CP),
                _full_spec(1, CP)],
      out_specs=_row_spec(CP),
      out_shape=jax.ShapeDtypeStruct((N, CP), jnp.float32),
  )(s30, s31, d0, d1, h2, wl3T, wr3T, bl3p)

  return outp[:, :C]
